# Initial kernel scaffold; baseline (speedup 1.0000x reference)
import jax
import jax.numpy as jnp
from jax.experimental import pallas as pl


def kernel(features, edge_index, edge_vals, kernel, bias, skip_weight):
    raise NotImplementedError("write your pallas kernel here")


# speedup vs baseline: 3.9875x; 3.9875x over previous
import functools

import jax
import jax.numpy as jnp
from jax import lax
from jax.experimental import pallas as pl
from jax.experimental.pallas import tpu as pltpu
from jax.experimental.pallas import tpu_sc as plsc

N = 10000
E = 320000
D = 128

NC = 2
NS = 16
NW = NC * NS

EPT = E // NW
KBLK = 80
NBLK = EPT // KBLK

NP = 10240
RPT = NP // NS
RCH = 128
NCHK = RPT // RCH

_MESH = plsc.VectorSubcoreMesh(core_axis_name="c", subcore_axis_name="s")


@functools.partial(
    pl.kernel,
    mesh=_MESH,
    out_type=jax.ShapeDtypeStruct((NC, NP, D), jnp.float32),
    scratch_types=[
        pltpu.VMEM((KBLK,), jnp.int32),
        pltpu.VMEM((KBLK,), jnp.int32),
        pltpu.VMEM((KBLK,), jnp.float32),
        pltpu.VMEM((KBLK, D), jnp.float32),
        pltpu.VMEM((RCH, D), jnp.float32),
        pltpu.VMEM_SHARED((NP, D), jnp.float32),
        pltpu.SemaphoreType.DMA,
    ],
    compiler_params=pltpu.CompilerParams(needs_layout_passes=False),
)
def _sc_scatter(feat, src, dst, ev, out, src_v, dst_v, ev_v, rows_v,
                stage_v, acc, sem):
    c = lax.axis_index("c")
    s = lax.axis_index("s")
    wid = s * NC + c

    def _zero(i, carry):
        r = i // 8
        j = i % 8
        stage_v[r, pl.ds(j * 16, 16)] = jnp.zeros((16,), jnp.float32)
        return carry

    lax.fori_loop(0, RCH * 8, _zero, 0)

    def _zcp(i, carry):
        pltpu.sync_copy(stage_v, acc.at[pl.ds(s * RPT + i * RCH, RCH)])
        return carry

    lax.fori_loop(0, NCHK, _zcp, 0)
    plsc.subcore_barrier()

    ebase = wid * EPT

    def _block(b, carry):
        base = ebase + b * KBLK
        pltpu.sync_copy(src.at[pl.ds(base, KBLK)], src_v)
        pltpu.sync_copy(dst.at[pl.ds(base, KBLK)], dst_v)
        pltpu.sync_copy(ev.at[pl.ds(base, KBLK)], ev_v)
        pltpu.async_copy(feat.at[src_v], rows_v, sem).wait()

        def _scale(k, carry2):
            evb = plsc.load_gather(ev_v, [jnp.full((16,), k, jnp.int32)])
            for j in range(8):
                sl = pl.ds(j * 16, 16)
                rows_v[k, sl] = rows_v[k, sl] * evb
            return carry2

        lax.fori_loop(0, KBLK, _scale, 0)
        pltpu.sync_copy(rows_v, acc.at[dst_v], add=True)
        return carry

    lax.fori_loop(0, NBLK, _block, 0)
    plsc.subcore_barrier()

    def _wb(i, carry):
        r0 = s * RPT + i * RCH
        pltpu.sync_copy(acc.at[pl.ds(r0, RCH)], stage_v)
        pltpu.sync_copy(stage_v, out.at[c, pl.ds(r0, RCH)])
        return carry

    lax.fori_loop(0, NCHK, _wb, 0)


RB = 1000


def _tc_body(x_ref, p0_ref, p1_ref, w_ref, b_ref, sk_ref, o_ref):
    w = w_ref[...]
    ws = w * sk_ref[...]
    agg = p0_ref[...] + p1_ref[...]
    r = jnp.dot(x_ref[...], ws, preferred_element_type=jnp.float32)
    r = r + jnp.dot(agg, w, preferred_element_type=jnp.float32)
    r = r + b_ref[...]
    alpha = 1.6732632423543772848170429916717
    scale = 1.0507009873554804934193349852946
    neg = alpha * (jnp.exp(jnp.minimum(r, 0.0)) - 1.0)
    o_ref[...] = scale * jnp.where(r > 0, r, neg)


def _tc_fused(x, p0, p1, w, bias2, skip2):
    return pl.pallas_call(
        _tc_body,
        grid=(N // RB,),
        in_specs=[
            pl.BlockSpec((RB, D), lambda i: (i, 0)),
            pl.BlockSpec((RB, D), lambda i: (i, 0)),
            pl.BlockSpec((RB, D), lambda i: (i, 0)),
            pl.BlockSpec((D, D), lambda i: (0, 0)),
            pl.BlockSpec((1, D), lambda i: (0, 0)),
            pl.BlockSpec((1, D), lambda i: (0, 0)),
        ],
        out_specs=pl.BlockSpec((RB, D), lambda i: (i, 0)),
        out_shape=jax.ShapeDtypeStruct((N, D), jnp.float32),
    )(x, p0, p1, w, bias2, skip2)


def kernel(features, edge_index, edge_vals, kernel, bias, skip_weight):
    src = edge_index[0]
    dst = edge_index[1]
    partial = _sc_scatter(features, src, dst, edge_vals)
    bias2 = bias.reshape(1, D)
    skip2 = skip_weight.reshape(1, D)
    return _tc_fused(features, partial[0], partial[1], kernel, bias2, skip2)


# speedup vs baseline: 9.0037x; 2.2580x over previous
import functools

import jax
import jax.numpy as jnp
from jax import lax
from jax.experimental import pallas as pl
from jax.experimental.pallas import tpu as pltpu
from jax.experimental.pallas import tpu_sc as plsc

N = 10000
E = 320000
D = 128

NC = 2
NS = 16
NW = NC * NS

EPT = E // NW
KBLK = 80
NBLK = EPT // KBLK
SUP = 25
NSUP = NBLK // SUP
SUPE = SUP * KBLK
NPAIR = (SUP - 1) // 2

NP = 10240
RPT = NP // NS
RCH = 64
NCHK = RPT // RCH

UNROLL = 4

_MESH = plsc.VectorSubcoreMesh(core_axis_name="c", subcore_axis_name="s")


@functools.partial(
    pl.kernel,
    mesh=_MESH,
    out_type=jax.ShapeDtypeStruct((NC, NP, D), jnp.float32),
    scratch_types=[
        pltpu.VMEM((2, SUP, KBLK), jnp.int32),
        pltpu.VMEM((2, SUP, KBLK), jnp.int32),
        pltpu.VMEM((2, SUPE), jnp.float32),
        pltpu.VMEM((2, KBLK, D), jnp.float32),
        pltpu.VMEM((RCH, D), jnp.float32),
        pltpu.VMEM_SHARED((NP, D), jnp.float32),
        pltpu.SemaphoreType.DMA,
        pltpu.SemaphoreType.DMA,
        pltpu.SemaphoreType.DMA,
        pltpu.SemaphoreType.DMA,
        pltpu.SemaphoreType.DMA,
    ],
    compiler_params=pltpu.CompilerParams(needs_layout_passes=False),
)
def _sc_scatter(feat, src4, dst4, ev3, out, src_i, dst_i, ev_v, rows,
                stage_v, acc, sem_g0, sem_g1, sem_s0, sem_s1, sem_i):
    c = lax.axis_index("c")
    s = lax.axis_index("s")
    wid = s * NC + c

    def _zero(i, carry):
        r = i // 8
        j = i % 8
        stage_v[r, pl.ds(j * 16, 16)] = jnp.zeros((16,), jnp.float32)
        return carry

    lax.fori_loop(0, RCH * 8, _zero, 0)

    def _zcp(i, carry):
        pltpu.sync_copy(stage_v, acc.at[pl.ds(s * RPT + i * RCH, RCH)])
        return carry

    lax.fori_loop(0, NCHK, _zcp, 0)

    pltpu.sync_copy(src4.at[wid, 0], src_i.at[0])
    pltpu.sync_copy(dst4.at[wid, 0], dst_i.at[0])
    pltpu.sync_copy(ev3.at[wid, 0], ev_v.at[0])
    plsc.subcore_barrier()

    def _scale(p, par, base):
        par16 = jnp.full((16,), par, jnp.int32)

        def _sc_u(q, carry):
            for u in range(UNROLL):
                k = q * UNROLL + u
                evb = plsc.load_gather(
                    ev_v, [par16, jnp.full((16,), base + k, jnp.int32)])
                for j in range(8):
                    sl = pl.ds(j * 16, 16)
                    rows[p, k, sl] = rows[p, k, sl] * evb
            return carry

        lax.fori_loop(0, KBLK // UNROLL, _sc_u, 0)

    def _gather_start(par, slot, p, sem):
        pltpu.async_copy(feat.at[src_i.at[par, slot]], rows.at[p], sem)

    def _gather_wait(par, slot, p, sem):
        pltpu.make_async_copy(
            feat.at[src_i.at[par, slot]], rows.at[p], sem).wait()

    def _scatter_start(par, slot, p, sem):
        pltpu.async_copy(rows.at[p], acc.at[dst_i.at[par, slot]], sem,
                         add=True)

    def _scatter_wait(par, slot, p, sem):
        pltpu.make_async_copy(
            rows.at[p], acc.at[dst_i.at[par, slot]], sem).wait()

    def _sup(sup, carry):
        par = sup % 2
        nxt = 1 - par

        @pl.when(sup + 1 < NSUP)
        def _():
            pltpu.async_copy(src4.at[wid, sup + 1], src_i.at[nxt], sem_i)
            pltpu.async_copy(dst4.at[wid, sup + 1], dst_i.at[nxt], sem_i)
            pltpu.async_copy(ev3.at[wid, sup + 1], ev_v.at[nxt], sem_i)

        _gather_start(par, 0, 0, sem_g0)

        def _pair(t, carry2):
            s0 = 2 * t
            _gather_wait(par, s0, 0, sem_g0)
            _gather_start(par, s0 + 1, 1, sem_g1)
            _scale(0, par, s0 * KBLK)
            _scatter_start(par, s0, 0, sem_s0)
            _gather_wait(par, s0 + 1, 1, sem_g1)
            _scatter_wait(par, s0, 0, sem_s0)
            _gather_start(par, s0 + 2, 0, sem_g0)
            _scale(1, par, (s0 + 1) * KBLK)
            _scatter_start(par, s0 + 1, 1, sem_s1)
            _scatter_wait(par, s0 + 1, 1, sem_s1)
            return carry2

        lax.fori_loop(0, NPAIR, _pair, 0)

        st = SUP - 1
        _gather_wait(par, st, 0, sem_g0)
        _scale(0, par, st * KBLK)
        pltpu.sync_copy(rows.at[0], acc.at[dst_i.at[par, st]], add=True)

        @pl.when(sup + 1 < NSUP)
        def _():
            pltpu.make_async_copy(
                src4.at[wid, sup + 1], src_i.at[nxt], sem_i).wait()
            pltpu.make_async_copy(
                dst4.at[wid, sup + 1], dst_i.at[nxt], sem_i).wait()
            pltpu.make_async_copy(
                ev3.at[wid, sup + 1], ev_v.at[nxt], sem_i).wait()

        return carry

    lax.fori_loop(0, NSUP, _sup, 0)
    plsc.subcore_barrier()

    def _wb(i, carry):
        r0 = s * RPT + i * RCH
        pltpu.sync_copy(acc.at[pl.ds(r0, RCH)], stage_v)
        pltpu.sync_copy(stage_v, out.at[c, pl.ds(r0, RCH)])
        return carry

    lax.fori_loop(0, NCHK, _wb, 0)


RB = 1000


def _tc_body(x_ref, p0_ref, p1_ref, w_ref, b_ref, sk_ref, o_ref):
    w = w_ref[...]
    ws = w * sk_ref[...]
    agg = p0_ref[...] + p1_ref[...]
    r = jnp.dot(x_ref[...], ws, preferred_element_type=jnp.float32)
    r = r + jnp.dot(agg, w, preferred_element_type=jnp.float32)
    r = r + b_ref[...]
    alpha = 1.6732632423543772848170429916717
    scale = 1.0507009873554804934193349852946
    neg = alpha * (jnp.exp(jnp.minimum(r, 0.0)) - 1.0)
    o_ref[...] = scale * jnp.where(r > 0, r, neg)


def _tc_fused(x, p0, p1, w, bias2, skip2):
    return pl.pallas_call(
        _tc_body,
        grid=(N // RB,),
        in_specs=[
            pl.BlockSpec((RB, D), lambda i: (i, 0)),
            pl.BlockSpec((RB, D), lambda i: (i, 0)),
            pl.BlockSpec((RB, D), lambda i: (i, 0)),
            pl.BlockSpec((D, D), lambda i: (0, 0)),
            pl.BlockSpec((1, D), lambda i: (0, 0)),
            pl.BlockSpec((1, D), lambda i: (0, 0)),
        ],
        out_specs=pl.BlockSpec((RB, D), lambda i: (i, 0)),
        out_shape=jax.ShapeDtypeStruct((N, D), jnp.float32),
    )(x, p0, p1, w, bias2, skip2)


def kernel(features, edge_index, edge_vals, kernel, bias, skip_weight):
    src4 = edge_index[0].reshape(NW, NSUP, SUP, KBLK)
    dst4 = edge_index[1].reshape(NW, NSUP, SUP, KBLK)
    ev3 = edge_vals.reshape(NW, NSUP, SUPE)
    partial = _sc_scatter(features, src4, dst4, ev3)
    bias2 = bias.reshape(1, D)
    skip2 = skip_weight.reshape(1, D)
    return _tc_fused(features, partial[0], partial[1], kernel, bias2, skip2)


# speedup vs baseline: 9.4528x; 1.0499x over previous
import functools

import jax
import jax.numpy as jnp
from jax import lax
from jax.experimental import pallas as pl
from jax.experimental.pallas import tpu as pltpu
from jax.experimental.pallas import tpu_sc as plsc

N = 10000
E = 320000
D = 128

NC = 2
NS = 16
NW = NC * NS

EPT = E // NW
KBLK = 80
NBLK = EPT // KBLK
SUP = 25
NSUP = NBLK // SUP
SUPE = SUP * KBLK
NPAIR = (SUP - 1) // 2

NP = 10240
RPT = NP // NS
RCH = 64
NCHK = RPT // RCH

UNROLL = 4

_MESH = plsc.VectorSubcoreMesh(core_axis_name="c", subcore_axis_name="s")


@functools.partial(
    pl.kernel,
    mesh=_MESH,
    out_type=jax.ShapeDtypeStruct((NC, NP, D), jnp.float32),
    scratch_types=[
        pltpu.VMEM((2, SUP, KBLK), jnp.int32),
        pltpu.VMEM((2, SUP, KBLK), jnp.int32),
        pltpu.VMEM((2, SUPE), jnp.float32),
        pltpu.VMEM((2, KBLK, D), jnp.float32),
        pltpu.VMEM((RCH, D), jnp.float32),
        pltpu.VMEM_SHARED((NP, D), jnp.float32),
        pltpu.SemaphoreType.DMA,
        pltpu.SemaphoreType.DMA,
        pltpu.SemaphoreType.DMA,
        pltpu.SemaphoreType.DMA,
        pltpu.SemaphoreType.DMA,
    ],
    compiler_params=pltpu.CompilerParams(needs_layout_passes=False),
)
def _sc_scatter(feat, src4, dst4, ev3, out, src_i, dst_i, ev_v, rows,
                stage_v, acc, sem_g0, sem_g1, sem_s0, sem_s1, sem_i):
    c = lax.axis_index("c")
    s = lax.axis_index("s")
    wid = s * NC + c

    def _zero(i, carry):
        r = i // 8
        j = i % 8
        stage_v[r, pl.ds(j * 16, 16)] = jnp.zeros((16,), jnp.float32)
        return carry

    lax.fori_loop(0, RCH * 8, _zero, 0)

    def _zcp(i, carry):
        pltpu.sync_copy(stage_v, acc.at[pl.ds(s * RPT + i * RCH, RCH)])
        return carry

    lax.fori_loop(0, NCHK, _zcp, 0)

    pltpu.sync_copy(src4.at[wid, 0], src_i.at[0])
    pltpu.sync_copy(dst4.at[wid, 0], dst_i.at[0])
    pltpu.sync_copy(ev3.at[wid, 0], ev_v.at[0])
    plsc.subcore_barrier()

    def _scale(p, par, base):
        par16 = jnp.full((16,), par, jnp.int32)

        @plsc.parallel_loop(0, KBLK, unroll=UNROLL)
        def _sc_u(k):
            evb = plsc.load_gather(
                ev_v, [par16, jnp.full((16,), base + k, jnp.int32)])
            for j in range(8):
                sl = pl.ds(j * 16, 16)
                rows[p, k, sl] = rows[p, k, sl] * evb

    def _gather_start(par, slot, p, sem):
        pltpu.async_copy(feat.at[src_i.at[par, slot]], rows.at[p], sem)

    def _gather_wait(par, slot, p, sem):
        pltpu.make_async_copy(
            feat.at[src_i.at[par, slot]], rows.at[p], sem).wait()

    def _scatter_start(par, slot, p, sem):
        pltpu.async_copy(rows.at[p], acc.at[dst_i.at[par, slot]], sem,
                         add=True)

    def _scatter_wait(par, slot, p, sem):
        pltpu.make_async_copy(
            rows.at[p], acc.at[dst_i.at[par, slot]], sem).wait()

    def _sup(sup, carry):
        par = sup % 2
        nxt = 1 - par

        @pl.when(sup + 1 < NSUP)
        def _():
            pltpu.async_copy(src4.at[wid, sup + 1], src_i.at[nxt], sem_i)
            pltpu.async_copy(dst4.at[wid, sup + 1], dst_i.at[nxt], sem_i)
            pltpu.async_copy(ev3.at[wid, sup + 1], ev_v.at[nxt], sem_i)

        _gather_start(par, 0, 0, sem_g0)

        def _pair(t, carry2):
            s0 = 2 * t
            _gather_wait(par, s0, 0, sem_g0)
            _gather_start(par, s0 + 1, 1, sem_g1)
            _scale(0, par, s0 * KBLK)
            _scatter_start(par, s0, 0, sem_s0)
            _gather_wait(par, s0 + 1, 1, sem_g1)
            _scatter_wait(par, s0, 0, sem_s0)
            _gather_start(par, s0 + 2, 0, sem_g0)
            _scale(1, par, (s0 + 1) * KBLK)
            _scatter_start(par, s0 + 1, 1, sem_s1)
            _scatter_wait(par, s0 + 1, 1, sem_s1)
            return carry2

        lax.fori_loop(0, NPAIR, _pair, 0)

        st = SUP - 1
        _gather_wait(par, st, 0, sem_g0)
        _scale(0, par, st * KBLK)
        pltpu.sync_copy(rows.at[0], acc.at[dst_i.at[par, st]], add=True)

        @pl.when(sup + 1 < NSUP)
        def _():
            pltpu.make_async_copy(
                src4.at[wid, sup + 1], src_i.at[nxt], sem_i).wait()
            pltpu.make_async_copy(
                dst4.at[wid, sup + 1], dst_i.at[nxt], sem_i).wait()
            pltpu.make_async_copy(
                ev3.at[wid, sup + 1], ev_v.at[nxt], sem_i).wait()

        return carry

    lax.fori_loop(0, NSUP, _sup, 0)
    plsc.subcore_barrier()

    def _wb(i, carry):
        r0 = s * RPT + i * RCH
        pltpu.sync_copy(acc.at[pl.ds(r0, RCH)], stage_v)
        pltpu.sync_copy(stage_v, out.at[c, pl.ds(r0, RCH)])
        return carry

    lax.fori_loop(0, NCHK, _wb, 0)


RB = 1000


def _tc_body(x_ref, p0_ref, p1_ref, w_ref, b_ref, sk_ref, o_ref):
    w = w_ref[...]
    ws = w * sk_ref[...]
    agg = p0_ref[...] + p1_ref[...]
    r = jnp.dot(x_ref[...], ws, preferred_element_type=jnp.float32)
    r = r + jnp.dot(agg, w, preferred_element_type=jnp.float32)
    r = r + b_ref[...]
    alpha = 1.6732632423543772848170429916717
    scale = 1.0507009873554804934193349852946
    neg = alpha * (jnp.exp(jnp.minimum(r, 0.0)) - 1.0)
    o_ref[...] = scale * jnp.where(r > 0, r, neg)


def _tc_fused(x, p0, p1, w, bias2, skip2):
    return pl.pallas_call(
        _tc_body,
        grid=(N // RB,),
        in_specs=[
            pl.BlockSpec((RB, D), lambda i: (i, 0)),
            pl.BlockSpec((RB, D), lambda i: (i, 0)),
            pl.BlockSpec((RB, D), lambda i: (i, 0)),
            pl.BlockSpec((D, D), lambda i: (0, 0)),
            pl.BlockSpec((1, D), lambda i: (0, 0)),
            pl.BlockSpec((1, D), lambda i: (0, 0)),
        ],
        out_specs=pl.BlockSpec((RB, D), lambda i: (i, 0)),
        out_shape=jax.ShapeDtypeStruct((N, D), jnp.float32),
    )(x, p0, p1, w, bias2, skip2)


def kernel(features, edge_index, edge_vals, kernel, bias, skip_weight):
    src4 = edge_index[0].reshape(NW, NSUP, SUP, KBLK)
    dst4 = edge_index[1].reshape(NW, NSUP, SUP, KBLK)
    ev3 = edge_vals.reshape(NW, NSUP, SUPE)
    partial = _sc_scatter(features, src4, dst4, ev3)
    bias2 = bias.reshape(1, D)
    skip2 = skip_weight.reshape(1, D)
    return _tc_fused(features, partial[0], partial[1], kernel, bias2, skip2)


# speedup vs baseline: 10.6399x; 1.1256x over previous
import functools

import jax
import jax.numpy as jnp
from jax import lax
from jax.experimental import pallas as pl
from jax.experimental.pallas import tpu as pltpu
from jax.experimental.pallas import tpu_sc as plsc

N = 10000
E = 320000
D = 128

NC = 2
NS = 16
NW = NC * NS

EPT = E // NW
KBLK = 40
NBLK = EPT // KBLK
SUP = 50
NSUP = NBLK // SUP
NQ = SUP // 4 - 1

NP = 10240
RPT = NP // NS
RCH = KBLK
NCHK = RPT // RCH

UNROLL = 4

_MESH = plsc.VectorSubcoreMesh(core_axis_name="c", subcore_axis_name="s")


@functools.partial(
    pl.kernel,
    mesh=_MESH,
    out_type=jax.ShapeDtypeStruct((NC, NP, D), jnp.float32),
    scratch_types=[
        pltpu.VMEM((SUP, KBLK), jnp.int32),
        pltpu.VMEM((SUP, KBLK), jnp.int32),
        pltpu.VMEM((1, SUP * KBLK), jnp.float32),
        pltpu.VMEM((4, KBLK, D), jnp.float32),
        pltpu.VMEM_SHARED((NP, D), jnp.float32),
        pltpu.SemaphoreType.DMA((4,)),
        pltpu.SemaphoreType.DMA((4,)),
    ],
    compiler_params=pltpu.CompilerParams(needs_layout_passes=False),
)
def _sc_scatter(feat, src4, dst4, ev4, out, src_i, dst_i, ev_v, rows,
                acc, sem_g, sem_s):
    c = lax.axis_index("c")
    s = lax.axis_index("s")
    wid = s * NC + c

    def _zero(i, carry):
        r = i // 8
        j = i % 8
        rows[0, r, pl.ds(j * 16, 16)] = jnp.zeros((16,), jnp.float32)
        return carry

    lax.fori_loop(0, RCH * 8, _zero, 0)

    def _zcp(i, carry):
        pltpu.sync_copy(rows.at[0], acc.at[pl.ds(s * RPT + i * RCH, RCH)])
        return carry

    lax.fori_loop(0, NCHK, _zcp, 0)

    plsc.subcore_barrier()

    def _scale(p, slot):
        zero16 = jnp.zeros((16,), jnp.int32)
        base = slot * KBLK

        @plsc.parallel_loop(0, KBLK, unroll=UNROLL)
        def _sc_u(k):
            evb = plsc.load_gather(
                ev_v, [zero16, jnp.full((16,), base + k, jnp.int32)])
            for j in range(8):
                sl = pl.ds(j * 16, 16)
                rows[p, k, sl] = rows[p, k, sl] * evb

    def _gather_start(slot, p):
        pltpu.async_copy(feat.at[src_i.at[slot]], rows.at[p], sem_g.at[p])

    def _gather_wait(slot, p):
        pltpu.make_async_copy(
            feat.at[src_i.at[slot]], rows.at[p], sem_g.at[p]).wait()

    def _scatter_start(slot, p):
        pltpu.async_copy(rows.at[p], acc.at[dst_i.at[slot]],
                         sem_s.at[p], add=True)

    def _scatter_wait(slot, p):
        pltpu.make_async_copy(
            rows.at[p], acc.at[dst_i.at[slot]], sem_s.at[p]).wait()

    def _stage_mid(slot, i):
        _scatter_wait(slot - 2, (i + 2) % 4)
        _gather_start(slot + 2, (i + 2) % 4)
        _gather_wait(slot, i % 4)
        _scale(i % 4, slot)
        _scatter_start(slot, i % 4)

    def _sup(sup, carry):
        pltpu.sync_copy(src4.at[wid, sup], src_i)
        pltpu.sync_copy(dst4.at[wid, sup], dst_i)
        pltpu.sync_copy(ev4.at[wid, sup], ev_v.at[0])

        _gather_start(0, 0)
        _gather_start(1, 1)

        for i in range(2):
            _gather_start(i + 2, i + 2)
            _gather_wait(i, i)
            _scale(i, i)
            _scatter_start(i, i)
        for i in range(2, 4):
            _scatter_wait(i - 2, (i + 2) % 4)
            _gather_start(i + 2, (i + 2) % 4)
            _gather_wait(i, i)
            _scale(i, i)
            _scatter_start(i, i)

        def _quad(q, carry2):
            base = 4 * (q + 1)
            for i in range(4):
                _stage_mid(base + i, i)
            return carry2

        lax.fori_loop(0, NQ, _quad, 0)

        for off in range(2):
            slot = SUP - 2 + off
            i = slot % 4
            _scatter_wait(slot - 2, (i + 2) % 4)
            _gather_wait(slot, i)
            _scale(i, slot)
            _scatter_start(slot, i)
        _scatter_wait(SUP - 2, (SUP - 2) % 4)
        _scatter_wait(SUP - 1, (SUP - 1) % 4)

        return carry

    lax.fori_loop(0, NSUP, _sup, 0)
    plsc.subcore_barrier()

    def _wb(i, carry):
        r0 = s * RPT + i * RCH
        pltpu.sync_copy(acc.at[pl.ds(r0, RCH)], rows.at[0])
        pltpu.sync_copy(rows.at[0], out.at[c, pl.ds(r0, RCH)])
        return carry

    lax.fori_loop(0, NCHK, _wb, 0)


RB = 1000


def _tc_body(x_ref, p0_ref, p1_ref, w_ref, b_ref, sk_ref, o_ref):
    w = w_ref[...]
    ws = w * sk_ref[...]
    agg = p0_ref[...] + p1_ref[...]
    r = jnp.dot(x_ref[...], ws, preferred_element_type=jnp.float32)
    r = r + jnp.dot(agg, w, preferred_element_type=jnp.float32)
    r = r + b_ref[...]
    alpha = 1.6732632423543772848170429916717
    scale = 1.0507009873554804934193349852946
    neg = alpha * (jnp.exp(jnp.minimum(r, 0.0)) - 1.0)
    o_ref[...] = scale * jnp.where(r > 0, r, neg)


def _tc_fused(x, p0, p1, w, bias2, skip2):
    return pl.pallas_call(
        _tc_body,
        grid=(N // RB,),
        in_specs=[
            pl.BlockSpec((RB, D), lambda i: (i, 0)),
            pl.BlockSpec((RB, D), lambda i: (i, 0)),
            pl.BlockSpec((RB, D), lambda i: (i, 0)),
            pl.BlockSpec((D, D), lambda i: (0, 0)),
            pl.BlockSpec((1, D), lambda i: (0, 0)),
            pl.BlockSpec((1, D), lambda i: (0, 0)),
        ],
        out_specs=pl.BlockSpec((RB, D), lambda i: (i, 0)),
        out_shape=jax.ShapeDtypeStruct((N, D), jnp.float32),
    )(x, p0, p1, w, bias2, skip2)


def kernel(features, edge_index, edge_vals, kernel, bias, skip_weight):
    src4 = edge_index[0].reshape(NW, NSUP, SUP, KBLK)
    dst4 = edge_index[1].reshape(NW, NSUP, SUP, KBLK)
    ev4 = edge_vals.reshape(NW, NSUP, SUP * KBLK)
    partial = _sc_scatter(features, src4, dst4, ev4)
    bias2 = bias.reshape(1, D)
    skip2 = skip_weight.reshape(1, D)
    return _tc_fused(features, partial[0], partial[1], kernel, bias2, skip2)


# speedup vs baseline: 10.7003x; 1.0057x over previous
import functools

import jax
import jax.numpy as jnp
from jax import lax
from jax.experimental import pallas as pl
from jax.experimental.pallas import tpu as pltpu
from jax.experimental.pallas import tpu_sc as plsc

N = 10000
E = 320000
D = 128

NC = 2
NS = 16
NW = NC * NS

EPT = E // NW
KBLK = 40
NBLK = EPT // KBLK
SUP = 50
NSUP = NBLK // SUP
NQ = SUP // 4 - 1

NP = 10240
RPT = NP // NS
NROWS = 4 * KBLK

UNROLL = 4

_MESH = plsc.VectorSubcoreMesh(core_axis_name="c", subcore_axis_name="s")


@functools.partial(
    pl.kernel,
    mesh=_MESH,
    out_type=jax.ShapeDtypeStruct((NC, NP, D), jnp.float32),
    scratch_types=[
        pltpu.VMEM((SUP, KBLK), jnp.int32),
        pltpu.VMEM((SUP, KBLK), jnp.int32),
        pltpu.VMEM((SUP, KBLK), jnp.float32),
        pltpu.VMEM((NROWS, D), jnp.float32),
        pltpu.VMEM_SHARED((NP, D), jnp.float32),
        pltpu.SemaphoreType.DMA((4,)),
        pltpu.SemaphoreType.DMA((4,)),
    ],
    compiler_params=pltpu.CompilerParams(needs_layout_passes=False),
)
def _sc_scatter(feat, src4, dst4, ev4, out, src_i, dst_i, ev_v, rows,
                acc, sem_g, sem_s):
    c = lax.axis_index("c")
    s = lax.axis_index("s")
    wid = s * NC + c

    @plsc.parallel_loop(0, NROWS * 8, unroll=8)
    def _zero(i):
        r = i // 8
        j = i % 8
        rows[r, pl.ds(j * 16, 16)] = jnp.zeros((16,), jnp.float32)

    for i in range(RPT // NROWS):
        pltpu.sync_copy(rows.at[pl.ds(0, NROWS)],
                        acc.at[pl.ds(s * RPT + i * NROWS, NROWS)])

    plsc.subcore_barrier()

    def _scale(p, slot, b):
        rvec = jnp.full((16,), slot, jnp.int32)

        @plsc.parallel_loop(0, KBLK, unroll=UNROLL)
        def _sc_u(k):
            evb = plsc.load_gather(
                ev_v, [rvec, jnp.full((16,), k, jnp.int32)])
            r = p * KBLK + k
            for j in range(8):
                sl = pl.ds(j * 16, 16)
                rows[r, sl] = rows[r, sl] * evb

    def _gather_start(slot, p, b):
        pltpu.async_copy(feat.at[src_i.at[slot]],
                         rows.at[pl.ds(p * KBLK, KBLK)], sem_g.at[p])

    def _gather_wait(slot, p, b):
        pltpu.make_async_copy(
            feat.at[src_i.at[slot]],
            rows.at[pl.ds(p * KBLK, KBLK)], sem_g.at[p]).wait()

    def _scatter_start(slot, p, b):
        pltpu.async_copy(rows.at[pl.ds(p * KBLK, KBLK)],
                         acc.at[dst_i.at[slot]], sem_s.at[p],
                         add=True)

    def _scatter_wait(slot, p, b):
        pltpu.make_async_copy(
            rows.at[pl.ds(p * KBLK, KBLK)],
            acc.at[dst_i.at[slot]], sem_s.at[p]).wait()

    def _stage_mid(slot, i, b):
        _scatter_wait(slot - 2, (i + 2) % 4, b)
        _gather_start(slot + 2, (i + 2) % 4, b)
        _gather_wait(slot, i % 4, b)
        _scale(i % 4, slot, b)
        _scatter_start(slot, i % 4, b)

    def _sup(sup, carry):
        b = 0
        pltpu.sync_copy(src4.at[wid, sup], src_i)
        pltpu.sync_copy(dst4.at[wid, sup], dst_i)
        pltpu.sync_copy(ev4.at[wid, sup], ev_v)

        _gather_start(0, 0, b)
        _gather_start(1, 1, b)

        for i in range(2):
            _gather_start(i + 2, i + 2, b)
            _gather_wait(i, i, b)
            _scale(i, i, b)
            _scatter_start(i, i, b)
        for i in range(2, 4):
            _scatter_wait(i - 2, (i + 2) % 4, b)
            _gather_start(i + 2, (i + 2) % 4, b)
            _gather_wait(i, i, b)
            _scale(i, i, b)
            _scatter_start(i, i, b)

        def _quad(q, carry2):
            base = 4 * (q + 1)
            for i in range(4):
                _stage_mid(base + i, i, b)
            return carry2

        lax.fori_loop(0, NQ, _quad, 0)

        for off in range(2):
            slot = SUP - 2 + off
            i = slot % 4
            _scatter_wait(slot - 2, (i + 2) % 4, b)
            _gather_wait(slot, i, b)
            _scale(i, slot, b)
            _scatter_start(slot, i, b)
        _scatter_wait(SUP - 2, (SUP - 2) % 4, b)
        _scatter_wait(SUP - 1, (SUP - 1) % 4, b)
        return carry

    lax.fori_loop(0, NSUP, _sup, 0)
    plsc.subcore_barrier()

    pltpu.sync_copy(acc.at[pl.ds(s * RPT, RPT)],
                    out.at[c, pl.ds(s * RPT, RPT)])


RB = 1000


def _tc1_body(x_ref, w_ref, b_ref, sk_ref, o_ref):
    ws = w_ref[...] * sk_ref[...]
    o_ref[...] = (
        jnp.dot(x_ref[...], ws, preferred_element_type=jnp.float32)
        + b_ref[...])


def _tc1(x, w, bias2, skip2):
    return pl.pallas_call(
        _tc1_body,
        grid=(N // RB,),
        in_specs=[
            pl.BlockSpec((RB, D), lambda i: (i, 0)),
            pl.BlockSpec((D, D), lambda i: (0, 0)),
            pl.BlockSpec((1, D), lambda i: (0, 0)),
            pl.BlockSpec((1, D), lambda i: (0, 0)),
        ],
        out_specs=pl.BlockSpec((RB, D), lambda i: (i, 0)),
        out_shape=jax.ShapeDtypeStruct((N, D), jnp.float32),
    )(x, w, bias2, skip2)


def _tc2_body(z_ref, p0_ref, p1_ref, w_ref, o_ref):
    agg = p0_ref[...] + p1_ref[...]
    r = z_ref[...] + jnp.dot(agg, w_ref[...],
                             preferred_element_type=jnp.float32)
    alpha = 1.6732632423543772848170429916717
    scale = 1.0507009873554804934193349852946
    neg = alpha * (jnp.exp(jnp.minimum(r, 0.0)) - 1.0)
    o_ref[...] = scale * jnp.where(r > 0, r, neg)


def _tc2(z, p0, p1, w):
    return pl.pallas_call(
        _tc2_body,
        grid=(N // RB,),
        in_specs=[
            pl.BlockSpec((RB, D), lambda i: (i, 0)),
            pl.BlockSpec((RB, D), lambda i: (i, 0)),
            pl.BlockSpec((RB, D), lambda i: (i, 0)),
            pl.BlockSpec((D, D), lambda i: (0, 0)),
        ],
        out_specs=pl.BlockSpec((RB, D), lambda i: (i, 0)),
        out_shape=jax.ShapeDtypeStruct((N, D), jnp.float32),
    )(z, p0, p1, w)


def kernel(features, edge_index, edge_vals, kernel, bias, skip_weight):
    src4 = edge_index[0].reshape(NW, NSUP, SUP, KBLK)
    dst4 = edge_index[1].reshape(NW, NSUP, SUP, KBLK)
    ev4 = edge_vals.reshape(NW, NSUP, SUP, KBLK)
    partial = _sc_scatter(features, src4, dst4, ev4)
    bias2 = bias.reshape(1, D)
    skip2 = skip_weight.reshape(1, D)
    z = _tc1(features, kernel, bias2, skip2)
    return _tc2(z, partial[0], partial[1], kernel)
